# baseline (speedup 1.0000x reference)
import functools

import jax
import jax.numpy as jnp
from jax import lax
from jax.experimental import pallas as pl
from jax.experimental.pallas import tpu as pltpu
from jax.experimental.pallas import tpu_sc as plsc

N = 10000
E = 160000
D_IN = 256
H = 512
D_OUT = 128
G = 64

NCH = 4
CH = H // NCH
NSUB = 16
NCORE = 2
EPT = 10240
EB = 32
NBATCH = EPT // EB
E_PAD = EPT * NSUB
NACC = N + 16
RPT = 624
TAIL = N - RPT * NSUB
BN = 1000
NBLK = N // BN


_MESH = plsc.VectorSubcoreMesh(core_axis_name="c", subcore_axis_name="s")


NBUF = 8
SLOT = 64
NSLOT = NBATCH // SLOT
NGRP = SLOT // NBUF


@functools.partial(
    pl.kernel,
    out_type=jax.ShapeDtypeStruct((NCH * N, CH), jnp.float32),
    mesh=_MESH,
    scratch_types=[
        pltpu.VMEM((SLOT, EB), jnp.int32),
        pltpu.VMEM((SLOT, EB), jnp.int32),
        [pltpu.VMEM((EB, CH), jnp.float32) for _ in range(NBUF)],
        pltpu.VMEM_SHARED((NACC, CH), jnp.float32),
        [pltpu.SemaphoreType.DMA for _ in range(NBUF)],
        [pltpu.SemaphoreType.DMA for _ in range(NBUF)],
    ],
)
def _sc_aggregate(h_hbm, src_hbm, dst_hbm, out_hbm, srcv, dstv, bufs, acc,
                  sems, ssems):
    core = lax.axis_index("c")
    sub = lax.axis_index("s")
    for j in range(NCH // NCORE):
        chunk = 2 * j + core
        row0 = chunk * N + sub * RPT
        pltpu.sync_copy(h_hbm.at[pl.ds(row0, RPT)], acc.at[pl.ds(sub * RPT, RPT)])

        @pl.when(sub == NSUB - 1)
        def _init_tail():
            pltpu.sync_copy(h_hbm.at[pl.ds(chunk * N + RPT * NSUB, TAIL)],
                            acc.at[pl.ds(RPT * NSUB, TAIL)])

        plsc.subcore_barrier()

        for slot in range(NSLOT):
            irow = (chunk * NSUB + sub) * NBATCH + slot * SLOT
            pltpu.sync_copy(src_hbm.at[pl.ds(irow, SLOT)], srcv)
            pltpu.sync_copy(dst_hbm.at[pl.ds(sub * NBATCH + slot * SLOT, SLOT)],
                            dstv)

            def _grp(m, carry):
                for k in range(NBUF):
                    @pl.when(m > 0)
                    def _drain_prev():
                        pltpu.make_async_copy(
                            bufs[k], acc.at[dstv.at[0]], ssems[k]).wait()
                    pltpu.async_copy(h_hbm.at[srcv.at[m * NBUF + k]], bufs[k],
                                     sems[k])
                for k in range(NBUF):
                    b = m * NBUF + k
                    pltpu.make_async_copy(
                        h_hbm.at[srcv.at[b]], bufs[k], sems[k]).wait()
                    pltpu.async_copy(bufs[k], acc.at[dstv.at[b]], ssems[k],
                                     add=True)
                return carry

            lax.fori_loop(0, NGRP, _grp, 0)
            for k in range(NBUF):
                pltpu.make_async_copy(
                    bufs[k], acc.at[dstv.at[0]], ssems[k]).wait()

        plsc.subcore_barrier()
        pltpu.sync_copy(acc.at[pl.ds(sub * RPT, RPT)], out_hbm.at[pl.ds(row0, RPT)])

        @pl.when(sub == NSUB - 1)
        def _out_tail():
            pltpu.sync_copy(acc.at[pl.ds(RPT * NSUB, TAIL)],
                            out_hbm.at[pl.ds(chunk * N + RPT * NSUB, TAIL)])

        plsc.subcore_barrier()


def _dot(a, b):
    return jnp.dot(a, b, preferred_element_type=jnp.float32)


def _pre_body(x_ref, w_ref, b_ref, out_ref):
    y = _dot(x_ref[...], w_ref[...]) + b_ref[...]
    for c in range(NCH):
        out_ref[c] = y[:, c * CH:(c + 1) * CH]


def _pre_mp(x, w, b):
    return pl.pallas_call(
        _pre_body,
        grid=(NBLK,),
        in_specs=[
            pl.BlockSpec((BN, D_IN), lambda i: (i, 0)),
            pl.BlockSpec((D_IN, H), lambda i: (0, 0)),
            pl.BlockSpec((1, H), lambda i: (0, 0)),
        ],
        out_specs=pl.BlockSpec((NCH, BN, CH), lambda i: (0, i, 0)),
        out_shape=jax.ShapeDtypeStruct((NCH, N, CH), jnp.float32),
    )(x, w, b)


def _mlp_body(z_ref, w1_ref, b1_ref, w2_ref, b2_ref, out_ref):
    z = jnp.concatenate([z_ref[c] for c in range(NCH)], axis=1)
    y = jnp.maximum(_dot(z, w1_ref[...]) + b1_ref[...], 0.0)
    o = jnp.maximum(_dot(y, w2_ref[...]) + b2_ref[...], 0.0)
    for c in range(NCH):
        out_ref[c] = o[:, c * CH:(c + 1) * CH]


def _mlp(z, w1, b1, w2, b2):
    return pl.pallas_call(
        _mlp_body,
        grid=(NBLK,),
        in_specs=[
            pl.BlockSpec((NCH, BN, CH), lambda i: (0, i, 0)),
            pl.BlockSpec((H, H), lambda i: (0, 0)),
            pl.BlockSpec((1, H), lambda i: (0, 0)),
            pl.BlockSpec((H, H), lambda i: (0, 0)),
            pl.BlockSpec((1, H), lambda i: (0, 0)),
        ],
        out_specs=pl.BlockSpec((NCH, BN, CH), lambda i: (0, i, 0)),
        out_shape=jax.ShapeDtypeStruct((NCH, N, CH), jnp.float32),
    )(z, w1, b1, w2, b2)


def _final_body(h_ref, bid_ref, w1_ref, b1_ref, w2_ref, b2_ref, out_ref, acc_ref):
    i = pl.program_id(0)

    @pl.when(i == 0)
    def _init():
        acc_ref[...] = jnp.zeros_like(acc_ref)

    bv = bid_ref[0, 0, :]
    onehot = (bv[:, None] == lax.broadcasted_iota(jnp.int32, (BN, G), 1)).astype(
        jnp.float32)
    hblk = jnp.concatenate([h_ref[c] for c in range(NCH)], axis=1)
    acc_ref[...] += lax.dot_general(
        onehot, hblk, (((0,), (0,)), ((), ())),
        preferred_element_type=jnp.float32)

    @pl.when(i == NBLK - 1)
    def _post():
        pooled = acc_ref[...]
        y = jnp.maximum(_dot(pooled, w1_ref[...]) + b1_ref[...], 0.0)
        o = _dot(y, w2_ref[...]) + b2_ref[...]
        m = jnp.max(o, axis=1, keepdims=True)
        lse = m + jnp.log(jnp.sum(jnp.exp(o - m), axis=1, keepdims=True))
        out_ref[...] = o - lse


def _pool_post(h, bid3, w1, b1, w2, b2):
    return pl.pallas_call(
        _final_body,
        grid=(NBLK,),
        in_specs=[
            pl.BlockSpec((NCH, BN, CH), lambda i: (0, i, 0)),
            pl.BlockSpec((1, 1, BN), lambda i: (i, 0, 0)),
            pl.BlockSpec((H, H), lambda i: (0, 0)),
            pl.BlockSpec((1, H), lambda i: (0, 0)),
            pl.BlockSpec((H, D_OUT), lambda i: (0, 0)),
            pl.BlockSpec((1, D_OUT), lambda i: (0, 0)),
        ],
        out_specs=pl.BlockSpec((G, D_OUT), lambda i: (0, 0)),
        out_shape=jax.ShapeDtypeStruct((G, D_OUT), jnp.float32),
        scratch_shapes=[pltpu.VMEM((G, H), jnp.float32)],
    )(h, bid3, w1, b1, w2, b2)


def kernel(x, edge_index, batch, pre_W, pre_b,
           conv0_W1, conv0_b1, conv0_W2, conv0_b2,
           conv1_W1, conv1_b1, conv1_W2, conv1_b2,
           conv2_W1, conv2_b1, conv2_W2, conv2_b2,
           post_W1, post_b1, post_W2, post_b2):
    src = edge_index[0]
    dst = edge_index[1]
    order = jnp.argsort(src)
    src = src[order]
    dst = dst[order]
    src4 = src[None, :] + (jnp.arange(NCH, dtype=jnp.int32) * N)[:, None]
    src_flat = jnp.pad(src4, ((0, 0), (0, E_PAD - E))).reshape(
        NCH * NSUB * NBATCH, EB)
    dst_pad = jnp.pad(dst, (0, E_PAD - E), constant_values=N).reshape(
        NSUB * NBATCH, EB)
    bid3 = batch.reshape(NBLK, 1, BN)

    h = _pre_mp(x, pre_W, pre_b.reshape(1, H))
    convs = [
        (conv0_W1, conv0_b1, conv0_W2, conv0_b2),
        (conv1_W1, conv1_b1, conv1_W2, conv1_b2),
        (conv2_W1, conv2_b1, conv2_W2, conv2_b2),
    ]
    for w1, b1, w2, b2 in convs:
        z2d = _sc_aggregate(h.reshape(NCH * N, CH), src_flat, dst_pad)
        h = _mlp(z2d.reshape(NCH, N, CH), w1, b1.reshape(1, H),
                 w2, b2.reshape(1, H))
    return _pool_post(h, bid3, post_W1, post_b1.reshape(1, H),
                      post_W2, post_b2.reshape(1, D_OUT))


# speedup vs baseline: 1.2101x; 1.2101x over previous
import functools

import jax
import jax.numpy as jnp
from jax import lax
from jax.experimental import pallas as pl
from jax.experimental.pallas import tpu as pltpu
from jax.experimental.pallas import tpu_sc as plsc

N = 10000
E = 160000
D_IN = 256
H = 512
D_OUT = 128
G = 64

NCH = 4
CH = H // NCH
NSUB = 16
NCORE = 2
EPT = 10240
EB = 128
NBATCH = EPT // EB
E_PAD = EPT * NSUB
NACC = N + 16
RPT = 624
TAIL = N - RPT * NSUB
BN = 1000
NBLK = N // BN


_MESH = plsc.VectorSubcoreMesh(core_axis_name="c", subcore_axis_name="s")


NBUF = 2
HB = NBATCH // 2
NGRP_H = HB // NBUF


@functools.partial(
    pl.kernel,
    out_type=jax.ShapeDtypeStruct((NCH * N, CH), jnp.float32),
    mesh=_MESH,
    scratch_types=[
        pltpu.VMEM((HB, EB), jnp.int32),
        pltpu.VMEM((NBATCH, EB), jnp.int32),
        [pltpu.VMEM((EB, CH), jnp.float32) for _ in range(NBUF)],
        pltpu.VMEM_SHARED((NACC, CH), jnp.float32),
        [pltpu.SemaphoreType.DMA for _ in range(NBUF)],
    ],
)
def _sc_aggregate(h_hbm, src_hbm, dst_hbm, out_hbm, srcv, dstv, bufs, acc,
                  sems):
    core = lax.axis_index("c")
    sub = lax.axis_index("s")
    for j in range(NCH // NCORE):
        chunk = 2 * j + core
        row0 = chunk * N + sub * RPT
        pltpu.sync_copy(h_hbm.at[pl.ds(row0, RPT)], acc.at[pl.ds(sub * RPT, RPT)])

        @pl.when(sub == NSUB - 1)
        def _init_tail():
            pltpu.sync_copy(h_hbm.at[pl.ds(chunk * N + RPT * NSUB, TAIL)],
                            acc.at[pl.ds(RPT * NSUB, TAIL)])

        plsc.subcore_barrier()

        pltpu.sync_copy(dst_hbm.at[pl.ds(sub * NBATCH, NBATCH)], dstv)

        for half in range(2):
            irow = (chunk * NSUB + sub) * NBATCH + half * HB
            pltpu.sync_copy(src_hbm.at[pl.ds(irow, HB)], srcv)
            b0 = half * HB

            for i in range(NBUF):
                pltpu.async_copy(h_hbm.at[srcv.at[i]], bufs[i], sems[i])

            def _group(g, carry):
                for i in range(NBUF):
                    b = g * NBUF + i
                    pltpu.make_async_copy(
                        h_hbm.at[srcv.at[b]], bufs[i], sems[i]).wait()
                    pltpu.sync_copy(bufs[i], acc.at[dstv.at[b0 + b]], add=True)

                    @pl.when(g < NGRP_H - 1)
                    def _refill():
                        pltpu.async_copy(
                            h_hbm.at[srcv.at[b + NBUF]], bufs[i], sems[i])

                return carry

            lax.fori_loop(0, NGRP_H, _group, 0)

        plsc.subcore_barrier()
        pltpu.sync_copy(acc.at[pl.ds(sub * RPT, RPT)], out_hbm.at[pl.ds(row0, RPT)])

        @pl.when(sub == NSUB - 1)
        def _out_tail():
            pltpu.sync_copy(acc.at[pl.ds(RPT * NSUB, TAIL)],
                            out_hbm.at[pl.ds(chunk * N + RPT * NSUB, TAIL)])

        plsc.subcore_barrier()


def _dot(a, b):
    return jnp.dot(a, b, preferred_element_type=jnp.float32)


def _pre_body(x_ref, w_ref, b_ref, out_ref):
    y = _dot(x_ref[...], w_ref[...]) + b_ref[...]
    for c in range(NCH):
        out_ref[c] = y[:, c * CH:(c + 1) * CH]


def _pre_mp(x, w, b):
    return pl.pallas_call(
        _pre_body,
        grid=(NBLK,),
        in_specs=[
            pl.BlockSpec((BN, D_IN), lambda i: (i, 0)),
            pl.BlockSpec((D_IN, H), lambda i: (0, 0)),
            pl.BlockSpec((1, H), lambda i: (0, 0)),
        ],
        out_specs=pl.BlockSpec((NCH, BN, CH), lambda i: (0, i, 0)),
        out_shape=jax.ShapeDtypeStruct((NCH, N, CH), jnp.float32),
    )(x, w, b)


def _mlp_body(z_ref, w1_ref, b1_ref, w2_ref, b2_ref, out_ref):
    z = jnp.concatenate([z_ref[c] for c in range(NCH)], axis=1)
    y = jnp.maximum(_dot(z, w1_ref[...]) + b1_ref[...], 0.0)
    o = jnp.maximum(_dot(y, w2_ref[...]) + b2_ref[...], 0.0)
    for c in range(NCH):
        out_ref[c] = o[:, c * CH:(c + 1) * CH]


def _mlp(z, w1, b1, w2, b2):
    return pl.pallas_call(
        _mlp_body,
        grid=(NBLK,),
        in_specs=[
            pl.BlockSpec((NCH, BN, CH), lambda i: (0, i, 0)),
            pl.BlockSpec((H, H), lambda i: (0, 0)),
            pl.BlockSpec((1, H), lambda i: (0, 0)),
            pl.BlockSpec((H, H), lambda i: (0, 0)),
            pl.BlockSpec((1, H), lambda i: (0, 0)),
        ],
        out_specs=pl.BlockSpec((NCH, BN, CH), lambda i: (0, i, 0)),
        out_shape=jax.ShapeDtypeStruct((NCH, N, CH), jnp.float32),
    )(z, w1, b1, w2, b2)


def _final_body(h_ref, bid_ref, w1_ref, b1_ref, w2_ref, b2_ref, out_ref, acc_ref):
    i = pl.program_id(0)

    @pl.when(i == 0)
    def _init():
        acc_ref[...] = jnp.zeros_like(acc_ref)

    bv = bid_ref[0, 0, :]
    onehot = (bv[:, None] == lax.broadcasted_iota(jnp.int32, (BN, G), 1)).astype(
        jnp.float32)
    hblk = jnp.concatenate([h_ref[c] for c in range(NCH)], axis=1)
    acc_ref[...] += lax.dot_general(
        onehot, hblk, (((0,), (0,)), ((), ())),
        preferred_element_type=jnp.float32)

    @pl.when(i == NBLK - 1)
    def _post():
        pooled = acc_ref[...]
        y = jnp.maximum(_dot(pooled, w1_ref[...]) + b1_ref[...], 0.0)
        o = _dot(y, w2_ref[...]) + b2_ref[...]
        m = jnp.max(o, axis=1, keepdims=True)
        lse = m + jnp.log(jnp.sum(jnp.exp(o - m), axis=1, keepdims=True))
        out_ref[...] = o - lse


def _pool_post(h, bid3, w1, b1, w2, b2):
    return pl.pallas_call(
        _final_body,
        grid=(NBLK,),
        in_specs=[
            pl.BlockSpec((NCH, BN, CH), lambda i: (0, i, 0)),
            pl.BlockSpec((1, 1, BN), lambda i: (i, 0, 0)),
            pl.BlockSpec((H, H), lambda i: (0, 0)),
            pl.BlockSpec((1, H), lambda i: (0, 0)),
            pl.BlockSpec((H, D_OUT), lambda i: (0, 0)),
            pl.BlockSpec((1, D_OUT), lambda i: (0, 0)),
        ],
        out_specs=pl.BlockSpec((G, D_OUT), lambda i: (0, 0)),
        out_shape=jax.ShapeDtypeStruct((G, D_OUT), jnp.float32),
        scratch_shapes=[pltpu.VMEM((G, H), jnp.float32)],
    )(h, bid3, w1, b1, w2, b2)


def kernel(x, edge_index, batch, pre_W, pre_b,
           conv0_W1, conv0_b1, conv0_W2, conv0_b2,
           conv1_W1, conv1_b1, conv1_W2, conv1_b2,
           conv2_W1, conv2_b1, conv2_W2, conv2_b2,
           post_W1, post_b1, post_W2, post_b2):
    src = edge_index[0]
    dst = edge_index[1]
    src4 = src[None, :] + (jnp.arange(NCH, dtype=jnp.int32) * N)[:, None]
    src_flat = jnp.pad(src4, ((0, 0), (0, E_PAD - E))).reshape(
        NCH * NSUB * NBATCH, EB)
    dst_pad = jnp.pad(dst, (0, E_PAD - E), constant_values=N).reshape(
        NSUB * NBATCH, EB)
    bid3 = batch.reshape(NBLK, 1, BN)

    h = _pre_mp(x, pre_W, pre_b.reshape(1, H))
    convs = [
        (conv0_W1, conv0_b1, conv0_W2, conv0_b2),
        (conv1_W1, conv1_b1, conv1_W2, conv1_b2),
        (conv2_W1, conv2_b1, conv2_W2, conv2_b2),
    ]
    for w1, b1, w2, b2 in convs:
        z2d = _sc_aggregate(h.reshape(NCH * N, CH), src_flat, dst_pad)
        h = _mlp(z2d.reshape(NCH, N, CH), w1, b1.reshape(1, H),
                 w2, b2.reshape(1, H))
    return _pool_post(h, bid3, post_W1, post_b1.reshape(1, H),
                      post_W2, post_b2.reshape(1, D_OUT))
